# baseline (speedup 1.0000x reference)
import functools

import jax
import jax.numpy as jnp
from jax import lax
from jax.experimental import pallas as pl
from jax.experimental.pallas import tpu as pltpu
from jax.experimental.pallas import tpu_sc as plsc

_NC, _NS = 2, 16
_NW = _NC * _NS
_W = 128
_BB = 8


def _pair_gather(table2, idx, n, d2):
    b_per_w = n // _NW
    chunks = b_per_w // _W
    mesh = plsc.VectorSubcoreMesh(core_axis_name="c", subcore_axis_name="s")

    @functools.partial(
        pl.kernel, mesh=mesh,
        out_type=jax.ShapeDtypeStruct((n, d2), jnp.float32),
        scratch_types=[
            pltpu.VMEM((chunks, _W), jnp.int32),
            pltpu.VMEM((_W, d2), jnp.float32),
            pltpu.SemaphoreType.DMA,
        ],
    )
    def _gather(table_hbm, idx_hbm, out_hbm, idx_v, rows_v, sem):
        wid = lax.axis_index("s") * _NC + lax.axis_index("c")
        base = wid * b_per_w
        pltpu.sync_copy(idx_hbm.at[wid], idx_v)

        @pl.loop(0, chunks)
        def _(g):
            pltpu.async_copy(table_hbm.at[idx_v.at[g]], rows_v, sem).wait()
            pltpu.sync_copy(rows_v, out_hbm.at[pl.ds(base + g * _W, _W)])

    return _gather(table2, idx)


def _half_select(pairs3, token_ids, batch, seq, d):
    def body(g_ref, p_ref, o_ref):
        g = g_ref[...]
        p = p_ref[...]
        o_ref[...] = jnp.where((p & 1)[:, :, None] == 1,
                               g[:, :, d:], g[:, :, :d])

    return pl.pallas_call(
        body,
        grid=(batch // _BB,),
        in_specs=[
            pl.BlockSpec((_BB, seq, 2 * d), lambda i: (i, 0, 0)),
            pl.BlockSpec((_BB, seq), lambda i: (i, 0)),
        ],
        out_specs=pl.BlockSpec((_BB, seq, d), lambda i: (i, 0, 0)),
        out_shape=jax.ShapeDtypeStruct((batch, seq, d), jnp.float32),
        compiler_params=pltpu.CompilerParams(
            dimension_semantics=("parallel",)),
    )(pairs3, token_ids)


def kernel(token_ids, embeddings):
    batch, seq = token_ids.shape
    vocab, d = embeddings.shape
    n = batch * seq
    b_per_w = n // _NW
    chunks = b_per_w // _W
    table2 = embeddings.reshape(vocab // 2, 2 * d)
    idx = (token_ids >> 1).reshape(_NW, chunks, _W)

    pairs = _pair_gather(table2, idx, n, 2 * d)
    pairs3 = pairs.reshape(batch, seq, 2 * d)
    return _half_select(pairs3, token_ids, batch, seq, d)


# speedup vs baseline: 1.0641x; 1.0641x over previous
import functools

import jax
import jax.numpy as jnp
from jax import lax
from jax.experimental import pallas as pl
from jax.experimental.pallas import tpu as pltpu
from jax.experimental.pallas import tpu_sc as plsc

_NC, _NS = 2, 16
_NW = _NC * _NS
_W = 128
_BLK = 2048


def _pair_gather(table2, idx, n, d2):
    b_per_w = n // _NW
    chunks = b_per_w // _W
    mesh = plsc.VectorSubcoreMesh(core_axis_name="c", subcore_axis_name="s")

    @functools.partial(
        pl.kernel, mesh=mesh,
        out_type=jax.ShapeDtypeStruct((n, d2), jnp.float32),
        scratch_types=[
            pltpu.VMEM((chunks, _W), jnp.int32),
            pltpu.VMEM((_W, d2), jnp.float32),
            pltpu.SemaphoreType.DMA,
        ],
    )
    def _gather(table_hbm, idx_hbm, out_hbm, idx_v, rows_v, sem):
        wid = lax.axis_index("s") * _NC + lax.axis_index("c")
        base = wid * b_per_w
        pltpu.sync_copy(idx_hbm.at[wid], idx_v)

        @pl.loop(0, chunks)
        def _(g):
            pltpu.async_copy(table_hbm.at[idx_v.at[g]], rows_v, sem).wait()
            pltpu.sync_copy(rows_v, out_hbm.at[pl.ds(base + g * _W, _W)])

    return _gather(table2, idx)


def _half_select(pairs, parity8, n, d):
    def body(g_ref, m_ref, o_ref):
        g = g_ref[...]
        m = m_ref[...] != 0
        o_ref[...] = jnp.where(m, g[:, d:], g[:, :d])

    return pl.pallas_call(
        body,
        grid=(n // _BLK,),
        in_specs=[
            pl.BlockSpec((_BLK, 2 * d), lambda i: (i, 0)),
            pl.BlockSpec((_BLK, 1), lambda i: (i, 0)),
        ],
        out_specs=pl.BlockSpec((_BLK, d), lambda i: (i, 0)),
        out_shape=jax.ShapeDtypeStruct((n, d), jnp.float32),
        compiler_params=pltpu.CompilerParams(
            dimension_semantics=("parallel",)),
    )(pairs, parity8)


def kernel(token_ids, embeddings):
    batch, seq = token_ids.shape
    vocab, d = embeddings.shape
    n = batch * seq
    b_per_w = n // _NW
    chunks = b_per_w // _W
    table2 = embeddings.reshape(vocab // 2, 2 * d)
    idx = (token_ids >> 1).reshape(_NW, chunks, _W)
    parity8 = (token_ids & 1).astype(jnp.int8).reshape(n, 1)

    pairs = _pair_gather(table2, idx, n, 2 * d)
    out = _half_select(pairs, parity8, n, d)
    return out.reshape(batch, seq, d)


# speedup vs baseline: 1.1154x; 1.0482x over previous
import functools

import jax
import jax.numpy as jnp
from jax import lax
from jax.experimental import pallas as pl
from jax.experimental.pallas import tpu as pltpu
from jax.experimental.pallas import tpu_sc as plsc

_NC, _NS = 2, 16
_NW = _NC * _NS
_W = 128
_BLK = 2048


def _pair_gather(table2, idx, n, d2):
    b_per_w = n // _NW
    chunks = b_per_w // _W
    mesh = plsc.VectorSubcoreMesh(core_axis_name="c", subcore_axis_name="s")

    @functools.partial(
        pl.kernel, mesh=mesh,
        out_type=jax.ShapeDtypeStruct((n, d2), jnp.float32),
        scratch_types=[
            pltpu.VMEM((chunks, _W), jnp.int32),
            pltpu.VMEM((_W, d2), jnp.float32),
            pltpu.VMEM((_W, d2), jnp.float32),
            pltpu.SemaphoreType.DMA,
            pltpu.SemaphoreType.DMA,
        ],
    )
    def _gather(table_hbm, idx_hbm, out_hbm, idx_v, rows0, rows1, sem0, sem1):
        wid = lax.axis_index("s") * _NC + lax.axis_index("c")
        base = wid * b_per_w
        pltpu.sync_copy(idx_hbm.at[wid], idx_v)
        bufs = (rows0, rows1)
        sems = (sem0, sem1)
        pltpu.async_copy(table_hbm.at[idx_v.at[0]], rows0, sem0)

        @pl.loop(0, chunks, step=2)
        def _(g):
            for b in range(2):
                k = g + b
                pltpu.make_async_copy(table_hbm.at[idx_v.at[k]],
                                      bufs[b], sems[b]).wait()

                @pl.when(k + 1 < chunks)
                def _():
                    pltpu.async_copy(table_hbm.at[idx_v.at[k + 1]],
                                     bufs[1 - b], sems[1 - b])

                pltpu.sync_copy(bufs[b], out_hbm.at[pl.ds(base + k * _W, _W)])

    return _gather(table2, idx)


def _half_select(pairs, parity8, n, d):
    def body(g_ref, m_ref, o_ref):
        g = g_ref[...]
        m = m_ref[...] != 0
        o_ref[...] = jnp.where(m, g[:, d:], g[:, :d])

    return pl.pallas_call(
        body,
        grid=(n // _BLK,),
        in_specs=[
            pl.BlockSpec((_BLK, 2 * d), lambda i: (i, 0)),
            pl.BlockSpec((_BLK, 1), lambda i: (i, 0)),
        ],
        out_specs=pl.BlockSpec((_BLK, d), lambda i: (i, 0)),
        out_shape=jax.ShapeDtypeStruct((n, d), jnp.float32),
        compiler_params=pltpu.CompilerParams(
            dimension_semantics=("parallel",)),
    )(pairs, parity8)


def kernel(token_ids, embeddings):
    batch, seq = token_ids.shape
    vocab, d = embeddings.shape
    n = batch * seq
    b_per_w = n // _NW
    chunks = b_per_w // _W
    table2 = embeddings.reshape(vocab // 2, 2 * d)
    idx = (token_ids >> 1).reshape(_NW, chunks, _W)
    parity8 = (token_ids & 1).astype(jnp.int8).reshape(n, 1)

    pairs = _pair_gather(table2, idx, n, 2 * d)
    out = _half_select(pairs, parity8, n, d)
    return out.reshape(batch, seq, d)
